# baseline (speedup 1.0000x reference)
import functools

import jax
import jax.numpy as jnp
from jax import lax
from jax.experimental import pallas as pl
from jax.experimental.pallas import tpu as pltpu
from jax.experimental.pallas import tpu_sc as plsc

N_NODES = 10000
N_EDGES = 320000
CH = 128

NC = 2
NS = 16
NW = NC * NS
CHUNK = 80
E_PER_TILE = N_EDGES // NW
NCH = E_PER_TILE // CHUNK
IB = 5
IBE = IB * CHUNK
NBLK = NCH // IB
GDEPTH = 3
N_PAD = 10240
ROWS_PER_TILE = N_PAD // NS
LANES = 16


_mesh = plsc.VectorSubcoreMesh(core_axis_name="c", subcore_axis_name="s")


@functools.partial(
    pl.kernel,
    out_type=jax.ShapeDtypeStruct((NC, N_PAD, CH), jnp.float32),
    mesh=_mesh,
    scratch_types=[
        pltpu.VMEM((3 * IBE,), jnp.int32),
        pltpu.VMEM((3 * IBE,), jnp.int32),
        pltpu.VMEM((IB, CHUNK), jnp.int32),
        pltpu.VMEM((4, CHUNK, CH), jnp.float32),
        pltpu.VMEM_SHARED((N_PAD, CH), jnp.float32),
        pltpu.SemaphoreType.DMA,
        pltpu.SemaphoreType.DMA,
    ],
)
def _sc_segment_sum(row_hbm, col_hbm, x_hbm, out_hbm,
                    rowv, colv, rowv2, rows, acc, isem, gsem):
    c = lax.axis_index("c")
    s = lax.axis_index("s")
    cbase = (c * NS + s) * E_PER_TILE

    with jax.named_scope("zero_init"):
        def _zero_row(i, carry):
            zero = jnp.zeros((LANES,), jnp.float32)
            for j in range(CH // LANES):
                rows[0, i, pl.ds(j * LANES, LANES)] = zero
            return carry
        lax.fori_loop(0, CHUNK, _zero_row, 0)
        for k in range(ROWS_PER_TILE // CHUNK):
            pltpu.sync_copy(rows.at[0],
                            acc.at[pl.ds(s * ROWS_PER_TILE + k * CHUNK, CHUNK)])
        plsc.subcore_barrier()

    edge_scope = jax.named_scope("edge_loop")
    edge_scope.__enter__()
    pltpu.async_copy(row_hbm.at[pl.ds(cbase, IBE)], rowv.at[pl.ds(0, IBE)],
                     isem)
    pltpu.async_copy(col_hbm.at[pl.ds(cbase, IBE)], colv.at[pl.ds(0, IBE)],
                     isem)
    pltpu.async_copy(row_hbm.at[pl.ds(cbase + IBE, IBE)],
                     rowv.at[pl.ds(IBE, IBE)], isem)
    pltpu.async_copy(col_hbm.at[pl.ds(cbase + IBE, IBE)],
                     colv.at[pl.ds(IBE, IBE)], isem)
    pltpu.make_async_copy(row_hbm.at[pl.ds(cbase, IBE)],
                          rowv.at[pl.ds(0, IBE)], isem).wait()
    pltpu.make_async_copy(col_hbm.at[pl.ds(cbase, IBE)],
                          colv.at[pl.ds(0, IBE)], isem).wait()
    for g in range(GDEPTH):
        pltpu.async_copy(x_hbm.at[colv.at[pl.ds(g * CHUNK, CHUNK)]],
                         rows.at[g], gsem)

    def _block(b, carry):
        boff = lax.rem(b, 3) * IBE

        @pl.when(b + 1 <= NBLK - 1)
        def _wait_next_idx():
            noff = lax.rem(b + 1, 3) * IBE
            pltpu.make_async_copy(row_hbm.at[pl.ds(cbase, IBE)],
                                  rowv.at[pl.ds(noff, IBE)], isem).wait()
            pltpu.make_async_copy(col_hbm.at[pl.ds(cbase, IBE)],
                                  colv.at[pl.ds(noff, IBE)], isem).wait()

        @pl.when(b + 2 <= NBLK - 1)
        def _prefetch_idx():
            nb = b + 2
            noff = lax.rem(nb, 3) * IBE
            pltpu.async_copy(row_hbm.at[pl.ds(cbase + nb * IBE, IBE)],
                             rowv.at[pl.ds(noff, IBE)], isem)
            pltpu.async_copy(col_hbm.at[pl.ds(cbase + nb * IBE, IBE)],
                             colv.at[pl.ds(noff, IBE)], isem)

        for j in range(IB):
            for k in range(CHUNK // LANES):
                rowv2[j, pl.ds(k * LANES, LANES)] = (
                    rowv[pl.ds(boff + j * CHUNK + k * LANES, LANES)])

        def _chunk(i, carry2):
            g = b * IB + i
            pltpu.make_async_copy(
                x_hbm.at[colv.at[pl.ds(boff + i * CHUNK, CHUNK)]],
                rows.at[lax.rem(g, 4)], gsem).wait()
            gg = g + GDEPTH

            @pl.when(gg <= NCH - 1)
            def _issue_gather():
                goff = (lax.rem(lax.div(gg, IB), 3) * IBE
                        + lax.rem(gg, IB) * CHUNK)
                pltpu.async_copy(
                    x_hbm.at[colv.at[pl.ds(goff, CHUNK)]],
                    rows.at[lax.rem(gg, 4)], gsem)

            pltpu.sync_copy(rows.at[lax.rem(g, 4)], acc.at[rowv2.at[i]],
                            add=True)
            return carry2
        lax.fori_loop(0, IB, _chunk, 0)
        return carry
    lax.fori_loop(0, NBLK, _block, 0)
    edge_scope.__exit__(None, None, None)

    with jax.named_scope("writeback"):
        plsc.subcore_barrier()
        pltpu.sync_copy(acc.at[pl.ds(s * ROWS_PER_TILE, ROWS_PER_TILE)],
                        out_hbm.at[c, pl.ds(s * ROWS_PER_TILE, ROWS_PER_TILE)])


TC_BLOCK = 2000


def _tc_matmul_body(p_ref, w_ref, o_ref):
    seg = p_ref[0] + p_ref[1]
    o_ref[...] = lax.dot_general(
        seg, w_ref[...], (((1,), (1,)), ((), ())),
        preferred_element_type=jnp.float32)


def kernel(x, edge_index, edge_attr, W_fc, W_edge, W_att):
    row = edge_index[0].astype(jnp.int32)
    col = edge_index[1].astype(jnp.int32)
    partials = _sc_segment_sum(row, col, x)
    z = pl.pallas_call(
        _tc_matmul_body,
        grid=(N_NODES // TC_BLOCK,),
        in_specs=[
            pl.BlockSpec((2, TC_BLOCK, CH), lambda i: (0, i, 0)),
            pl.BlockSpec((CH, CH), lambda i: (0, 0)),
        ],
        out_specs=pl.BlockSpec((TC_BLOCK, CH), lambda i: (i, 0)),
        out_shape=jax.ShapeDtypeStruct((N_NODES, CH), jnp.float32),
    )(partials, W_fc)
    return z
